# baseline (speedup 1.0000x reference)
import functools

import jax
import jax.numpy as jnp
from jax import lax
from jax.experimental import pallas as pl
from jax.experimental.pallas import tpu as pltpu
from jax.experimental.pallas import tpu_sc as plsc


def _sc_gather_rows(table, idx):
    bsz = idx.shape[0]
    cols = table.shape[1]
    info = plsc.get_sparse_core_info()
    n_cores, n_sub = info.num_cores, info.num_subcores
    n_workers = n_cores * n_sub
    b_per_w = bsz // n_workers
    mesh = plsc.VectorSubcoreMesh(core_axis_name="c", subcore_axis_name="s")

    @functools.partial(
        pl.kernel,
        mesh=mesh,
        out_type=jax.ShapeDtypeStruct((bsz, cols), jnp.float32),
        scratch_types=[
            pltpu.VMEM((b_per_w,), jnp.int32),
            pltpu.VMEM((b_per_w, cols), jnp.float32),
            pltpu.SemaphoreType.DMA,
        ],
    )
    def gather_kernel(table_hbm, idx_hbm, out_hbm, idx_v, rows_v, sem):
        wid = lax.axis_index("s") * n_cores + lax.axis_index("c")
        base = wid * b_per_w
        pltpu.sync_copy(idx_hbm.at[pl.ds(base, b_per_w)], idx_v)
        pltpu.async_copy(table_hbm.at[idx_v], rows_v, sem).wait()
        pltpu.sync_copy(rows_v, out_hbm.at[pl.ds(base, b_per_w)])

    return gather_kernel(table, idx)


def _film_body(rows_ref, feat_ref, out_ref):
    g = rows_ref[:, :128]
    b = rows_ref[:, 128:]
    out_ref[...] = feat_ref[...] * g[:, None, :] + b[:, None, :]


def kernel(feat, alt_idx, gamma, beta):
    bsz, seq, dim = feat.shape
    flat = seq * dim
    n_lane_rows = flat // 128

    table = jnp.concatenate([gamma, gamma, beta, beta], axis=1)
    rows = _sc_gather_rows(table, alt_idx.astype(jnp.int32))

    feat3 = feat.reshape(bsz, n_lane_rows, 128)
    blk_b = 128
    film = pl.pallas_call(
        _film_body,
        grid=(bsz // blk_b,),
        in_specs=[
            pl.BlockSpec((blk_b, 256), lambda i: (i, 0)),
            pl.BlockSpec((blk_b, n_lane_rows, 128), lambda i: (i, 0, 0)),
        ],
        out_specs=pl.BlockSpec((blk_b, n_lane_rows, 128), lambda i: (i, 0, 0)),
        out_shape=jax.ShapeDtypeStruct((bsz, n_lane_rows, 128), jnp.float32),
    )
    out3 = film(rows, feat3)
    return out3.reshape(bsz, seq, dim)
